# baseline (device time: 417896 ns/iter reference)
import jax
import jax.numpy as jnp
from jax import lax
from jax.experimental import pallas as pl
from jax.experimental.pallas import tpu as pltpu

W = 32


def kernel(x, w_mat, scale_x, scale_w):
    m, k_sh = x.shape
    _, n = w_mat.shape
    mc = m // W

    def body(x_ref, w_ref, sx_ref, sw_ref, out_ref,
             acc_ref, recv_ref, send_sems, recv_sems, credit_sem):
        my = lax.axis_index("i")
        left = lax.rem(my + W - 1, W)
        right = lax.rem(my + 1, W)

        barrier = pltpu.get_barrier_semaphore()
        for nbr in (left, right):
            pl.semaphore_signal(
                barrier, inc=1,
                device_id=(nbr,), device_id_type=pl.DeviceIdType.MESH,
            )
        pl.semaphore_wait(barrier, 2)

        def pchunk(c):
            xc = x_ref[pl.ds(c * mc, mc), :]
            return lax.dot_general(
                xc, w_ref[:, :],
                dimension_numbers=(((1,), (0,)), ((), ())),
                preferred_element_type=jnp.int32,
            )

        def make_rdma(slot):
            return pltpu.make_async_remote_copy(
                src_ref=acc_ref.at[slot],
                dst_ref=recv_ref.at[slot],
                send_sem=send_sems.at[slot],
                recv_sem=recv_sems.at[slot],
                device_id=(right,),
                device_id_type=pl.DeviceIdType.MESH,
            )

        acc_ref[0] = pchunk(lax.rem(my + W - 1, W))
        rdmas = [make_rdma(0)]
        rdmas[0].start()

        for h in range(W - 1):
            c = lax.rem(my + 2 * W - 2 - h, W)
            p = pchunk(c)
            rdmas[h].wait_recv()
            total = recv_ref[h % 2] + p
            if h < W - 2:
                if h >= 1:
                    rdmas[h - 1].wait_send()
                acc_ref[(h + 1) % 2] = total
                if h >= 1:
                    pl.semaphore_wait(credit_sem, 1)
                r = make_rdma((h + 1) % 2)
                r.start()
                rdmas.append(r)
            else:
                out_ref[:, :] = (
                    total.astype(jnp.float32) * (sx_ref[0] * sw_ref[0])
                )
            if h <= W - 4:
                pl.semaphore_signal(
                    credit_sem, inc=1,
                    device_id=(left,), device_id_type=pl.DeviceIdType.MESH,
                )

        rdmas[W - 3].wait_send()
        rdmas[W - 2].wait_send()

    return pl.pallas_call(
        body,
        out_shape=jax.ShapeDtypeStruct((mc, n), jnp.float32),
        in_specs=[
            pl.BlockSpec(memory_space=pltpu.VMEM),
            pl.BlockSpec(memory_space=pltpu.VMEM),
            pl.BlockSpec(memory_space=pltpu.SMEM),
            pl.BlockSpec(memory_space=pltpu.SMEM),
        ],
        out_specs=pl.BlockSpec(memory_space=pltpu.VMEM),
        scratch_shapes=[
            pltpu.VMEM((2, mc, n), jnp.int32),
            pltpu.VMEM((2, mc, n), jnp.int32),
            pltpu.SemaphoreType.DMA((2,)),
            pltpu.SemaphoreType.DMA((2,)),
            pltpu.SemaphoreType.REGULAR,
        ],
        compiler_params=pltpu.CompilerParams(collective_id=0),
    )(x, w_mat, scale_x, scale_w)
